# baseline (speedup 1.0000x reference)
import jax
import jax.numpy as jnp
import numpy as np
from jax import lax
from jax.experimental import pallas as pl
from jax.experimental.pallas import tpu as pltpu
from jax.experimental.pallas import tpu_sc as plsc

N = 32 * 192 * 56 * 56
RANK = int(N * 0.9995)
NC, NS, L = 2, 16, 16
NW = NC * NS
PER_W = N // NW
CHUNK = 2048
PAIRS = PER_W // (2 * CHUNK)
VECS = CHUNK // L
K = 8
NB12 = 4096
NB3 = 256
ROWS_Q = 8
MIN32 = np.int32(-(2 ** 31))

_CP = pltpu.CompilerParams(needs_layout_passes=False)


def _mesh():
    return plsc.VectorSubcoreMesh(core_axis_name="c", subcore_axis_name="s")


def _key_u32(xv):
    u = plsc.bitcast(xv, jnp.int32)
    sgn = u >> 31
    return plsc.bitcast(u ^ (sgn | MIN32), jnp.uint32)


def _reduce_rows(h_hbm, prev_nb, hbuf, acc):
    for q in range(NW // ROWS_Q):
        pltpu.sync_copy(h_hbm.at[pl.ds(q * ROWS_Q * prev_nb, ROWS_Q * prev_nb)],
                        hbuf)

        @pl.loop(0, prev_nb // (2 * L))
        def _(cj):
            avals = []
            for g in range(2):
                ci = cj * 2 + g
                a = hbuf[pl.ds(ci * L, L)]
                for r in range(1, ROWS_Q):
                    a = a + hbuf[pl.ds(r * prev_nb + ci * L, L)]
                if q > 0:
                    a = a + acc[pl.ds(ci * L, L)]
                avals.append(a)
            for g in range(2):
                acc[pl.ds((cj * 2 + g) * L, L)] = avals[g]


def _scan_select(acc, nbins, rank):
    def step(i, carry):
        bcnt, cbelow, cum = carry
        v = acc[pl.ds(i * L, L)]
        cs = plsc.cumsum(v) + cum
        m = cs < rank
        bcnt = bcnt + jnp.sum(m.astype(jnp.int32))
        cbelow = cbelow + jnp.sum(jnp.where(m, v, 0))
        cum = cum + jnp.sum(v)
        return bcnt, cbelow, cum

    z = jnp.int32(0)
    bcnt, cbelow, _ = lax.fori_loop(0, nbins // L, step, (z, z, z))
    return bcnt, cbelow


def _make_hist(shift, nb, merge=None):
    stride = nb + 1
    scratch = [
        pltpu.VMEM((CHUNK,), jnp.float32),
        pltpu.VMEM((CHUNK,), jnp.float32),
        pltpu.VMEM((L * stride,), jnp.int32),
        pltpu.VMEM((nb,), jnp.int32),
        pltpu.SemaphoreType.DMA,
        pltpu.SemaphoreType.DMA,
    ]
    if merge is not None:
        prev_nb, match_shift, out_shift = merge
        scratch += [
            pltpu.VMEM((ROWS_Q * prev_nb,), jnp.int32),
            pltpu.VMEM((prev_nb,), jnp.int32),
            pltpu.VMEM((L,), jnp.uint32),
            pltpu.VMEM((L,), jnp.int32),
        ]
        out_type = (jax.ShapeDtypeStruct((NW * nb,), jnp.int32),
                    jax.ShapeDtypeStruct((L,), jnp.uint32),
                    jax.ShapeDtypeStruct((L,), jnp.int32))
    else:
        scratch += [pltpu.SemaphoreType.DMA]
        out_type = (jax.ShapeDtypeStruct((NW * nb,), jnp.int32),
                    jax.ShapeDtypeStruct((N,), jnp.float32))

    def body(x_hbm, *rest):
        if merge is not None:
            h_hbm, selp_hbm, rankp_hbm, out_hbm, selo_hbm, ranko_hbm = rest[:6]
            buf0, buf1, hist, orow, sem0, sem1, hbuf, acc, selv, rv = rest[6:]
        else:
            out_hbm, xc_hbm = rest[0], rest[1]
            buf0, buf1, hist, orow, sem0, sem1, wsem = rest[2:]
        wid = lax.axis_index("s") * NC + lax.axis_index("c")
        base = wid * PER_W

        if merge is None:
            pltpu.async_copy(x_hbm.at[pl.ds(base, PER_W)],
                             xc_hbm.at[pl.ds(base, PER_W)], wsem)

        def start(c, buf, sem):
            pltpu.async_copy(x_hbm.at[pl.ds(base + c * CHUNK, CHUNK)], buf, sem)

        def wait(buf, sem):
            pltpu.make_async_copy(x_hbm.at[pl.ds(0, CHUNK)], buf, sem).wait()

        start(0, buf0, sem0)
        start(1, buf1, sem1)

        if merge is not None:
            _reduce_rows(h_hbm, prev_nb, hbuf, acc)
            pltpu.sync_copy(selp_hbm, selv)
            pltpu.sync_copy(rankp_hbm, rv)
            selp = jnp.max(plsc.bitcast(selv[...], jnp.int32))
            rankp = jnp.max(rv[...])
            bcnt, cbelow = _scan_select(acc, prev_nb, rankp)
            sel_s = (selp << out_shift) | bcnt
            rank_s = rankp - cbelow
            sel = plsc.bitcast(jnp.full((L,), sel_s, jnp.int32), jnp.uint32)

            @pl.when(wid == 0)
            def _():
                selv[...] = sel
                rv[...] = jnp.full((L,), rank_s, jnp.int32)
                pltpu.sync_copy(selv, selo_hbm)
                pltpu.sync_copy(rv, ranko_hbm)

        zero16 = jnp.zeros((L,), jnp.int32)

        @pl.loop(0, stride, unroll=8)
        def _(i):
            hist[pl.ds(i * L, L)] = zero16

        lane_base = lax.iota(jnp.int32, L) * stride
        ones = jnp.ones((L,), jnp.int32)

        def process(buf):
            @pl.loop(0, VECS // K)
            def _(g):
                b0 = g * (K * L)
                keys = [_key_u32(buf[pl.ds(b0 + k * L, L)]) for k in range(K)]
                if shift:
                    bs = [(key >> shift) & jnp.uint32(nb - 1) for key in keys]
                else:
                    bs = [key & jnp.uint32(nb - 1) for key in keys]
                addrs = [lane_base + plsc.bitcast(b, jnp.int32) for b in bs]
                if merge is None:
                    for addr in addrs:
                        plsc.addupdate_scatter(hist, [addr], ones)
                else:
                    ms = [(key >> match_shift) == sel for key in keys]
                    for addr, m in zip(addrs, ms):
                        plsc.addupdate_scatter(hist, [addr], ones, mask=m)

        @pl.loop(0, PAIRS)
        def _(j):
            c0 = j * 2
            wait(buf0, sem0)
            process(buf0)

            @pl.when(j < PAIRS - 1)
            def _():
                start(c0 + 2, buf0, sem0)

            wait(buf1, sem1)
            process(buf1)

            @pl.when(j < PAIRS - 1)
            def _():
                start(c0 + 3, buf1, sem1)

        @pl.loop(0, nb // L)
        def _(ci):
            a = hist[pl.ds(ci * L, L)]
            for lane in range(1, L):
                a = a + hist[pl.ds(lane * stride + ci * L, L)]
            orow[pl.ds(ci * L, L)] = a

        pltpu.sync_copy(orow, out_hbm.at[pl.ds(wid * nb, nb)])

        if merge is None:
            pltpu.make_async_copy(x_hbm.at[pl.ds(base, PER_W)],
                                  xc_hbm.at[pl.ds(base, PER_W)], wsem).wait()

    return pl.kernel(body, out_type=out_type, mesh=_mesh(),
                     scratch_types=scratch, compiler_params=_CP)


def _make_final(nb, out_shift):
    scratch = [
        pltpu.VMEM((ROWS_Q * nb,), jnp.int32),
        pltpu.VMEM((nb,), jnp.int32),
        pltpu.VMEM((L,), jnp.uint32),
        pltpu.VMEM((L,), jnp.int32),
        pltpu.VMEM((L,), jnp.float32),
    ]
    out_type = jax.ShapeDtypeStruct((L,), jnp.float32)

    def body(h_hbm, selp_hbm, rankp_hbm, out_hbm, hbuf, acc, selv, rv, ov):
        wid = lax.axis_index("s") * NC + lax.axis_index("c")

        @pl.when(wid == 0)
        def _():
            _reduce_rows(h_hbm, nb, hbuf, acc)
            pltpu.sync_copy(selp_hbm, selv)
            pltpu.sync_copy(rankp_hbm, rv)
            selp = jnp.max(plsc.bitcast(selv[...], jnp.int32))
            rankp = jnp.max(rv[...])
            bcnt, _ = _scan_select(acc, nb, rankp)
            keyv = ((jnp.full((L,), selp, jnp.int32) << out_shift)
                    | jnp.full((L,), bcnt, jnp.int32))
            uv = jnp.where(keyv < 0, keyv ^ MIN32, ~keyv)
            ov[...] = plsc.bitcast(uv, jnp.float32)
            pltpu.sync_copy(ov, out_hbm)

    return pl.kernel(body, out_type=out_type, mesh=_mesh(),
                     scratch_types=scratch, compiler_params=_CP)


_hist1 = _make_hist(shift=20, nb=NB12)
_hist2 = _make_hist(shift=8, nb=NB12, merge=(NB12, 20, 12))
_hist3 = _make_hist(shift=0, nb=NB3, merge=(NB12, 8, 12))
_final = _make_final(nb=NB3, out_shift=8)


def kernel(x):
    xf = x.reshape(-1)
    sel0 = jnp.zeros((L,), jnp.uint32)
    r0 = jnp.full((L,), RANK, jnp.int32)
    h1, xc = _hist1(xf)
    h2, sel1, r1 = _hist2(xf, h1, sel0, r0)
    h3, sel2, r2 = _hist3(xf, h2, sel1, r1)
    v = _final(h3, sel2, r2)
    return xc.reshape(x.shape), v[0]


# speedup vs baseline: 4.6860x; 4.6860x over previous
import jax
import jax.numpy as jnp
import numpy as np
from jax import lax
from jax.experimental import pallas as pl
from jax.experimental.pallas import tpu as pltpu
from jax.experimental.pallas import tpu_sc as plsc

N = 32 * 192 * 56 * 56
RANK = int(N * 0.9995)
NC, NS, L = 2, 16, 16
NW = NC * NS
PER_W = N // NW
CHUNK = 2048
PAIRS = PER_W // (2 * CHUNK)
VECS = CHUNK // L
K = 8
NB12 = 4096
NB3 = 256
ROWS_Q = 8
MIN32 = np.int32(-(2 ** 31))

_CP = pltpu.CompilerParams(needs_layout_passes=False)


def _mesh():
    return plsc.VectorSubcoreMesh(core_axis_name="c", subcore_axis_name="s")


def _key_u32(xv):
    u = plsc.bitcast(xv, jnp.int32)
    sgn = u >> 31
    return plsc.bitcast(u ^ (sgn | MIN32), jnp.uint32)


def _reduce_rows(h_hbm, prev_nb, hbuf, acc):
    for q in range(NW // ROWS_Q):
        pltpu.sync_copy(h_hbm.at[pl.ds(q * ROWS_Q * prev_nb, ROWS_Q * prev_nb)],
                        hbuf)

        @pl.loop(0, prev_nb // (2 * L))
        def _(cj):
            avals = []
            for g in range(2):
                ci = cj * 2 + g
                a = hbuf[pl.ds(ci * L, L)]
                for r in range(1, ROWS_Q):
                    a = a + hbuf[pl.ds(r * prev_nb + ci * L, L)]
                if q > 0:
                    a = a + acc[pl.ds(ci * L, L)]
                avals.append(a)
            for g in range(2):
                acc[pl.ds((cj * 2 + g) * L, L)] = avals[g]


def _scan_select(acc, nbins, rank):
    def step(i, carry):
        bcnt, cbelow, cum = carry
        v = acc[pl.ds(i * L, L)]
        cs = plsc.cumsum(v) + cum
        m = cs < rank
        bcnt = bcnt + jnp.sum(m.astype(jnp.int32))
        cbelow = cbelow + jnp.sum(jnp.where(m, v, 0))
        cum = cum + jnp.sum(v)
        return bcnt, cbelow, cum

    z = jnp.int32(0)
    bcnt, cbelow, _ = lax.fori_loop(0, nbins // L, step, (z, z, z))
    return bcnt, cbelow


def _make_hist(shift, nb, merge=None):
    stride = nb + 1
    scratch = [
        pltpu.VMEM((CHUNK,), jnp.float32),
        pltpu.VMEM((CHUNK,), jnp.float32),
        pltpu.VMEM((L * stride,), jnp.int32),
        pltpu.VMEM((nb,), jnp.int32),
        pltpu.SemaphoreType.DMA,
        pltpu.SemaphoreType.DMA,
    ]
    if merge is not None:
        prev_nb, match_shift, out_shift = merge
        scratch += [
            pltpu.VMEM((ROWS_Q * prev_nb,), jnp.int32),
            pltpu.VMEM((prev_nb,), jnp.int32),
            pltpu.VMEM((L,), jnp.uint32),
            pltpu.VMEM((L,), jnp.int32),
        ]
        out_type = (jax.ShapeDtypeStruct((NW * nb,), jnp.int32),
                    jax.ShapeDtypeStruct((L,), jnp.uint32),
                    jax.ShapeDtypeStruct((L,), jnp.int32))
    else:
        out_type = jax.ShapeDtypeStruct((NW * nb,), jnp.int32)

    def body(x_hbm, *rest):
        if merge is not None:
            h_hbm, selp_hbm, rankp_hbm, out_hbm, selo_hbm, ranko_hbm = rest[:6]
            buf0, buf1, hist, orow, sem0, sem1, hbuf, acc, selv, rv = rest[6:]
        else:
            out_hbm = rest[0]
            buf0, buf1, hist, orow, sem0, sem1 = rest[1:]
        wid = lax.axis_index("s") * NC + lax.axis_index("c")
        base = wid * PER_W

        def start(c, buf, sem):
            pltpu.async_copy(x_hbm.at[pl.ds(base + c * CHUNK, CHUNK)], buf, sem)

        def wait(buf, sem):
            pltpu.make_async_copy(x_hbm.at[pl.ds(0, CHUNK)], buf, sem).wait()

        start(0, buf0, sem0)
        start(1, buf1, sem1)

        if merge is not None:
            _reduce_rows(h_hbm, prev_nb, hbuf, acc)
            pltpu.sync_copy(selp_hbm, selv)
            pltpu.sync_copy(rankp_hbm, rv)
            selp = jnp.max(plsc.bitcast(selv[...], jnp.int32))
            rankp = jnp.max(rv[...])
            bcnt, cbelow = _scan_select(acc, prev_nb, rankp)
            sel_s = (selp << out_shift) | bcnt
            rank_s = rankp - cbelow
            sel = plsc.bitcast(jnp.full((L,), sel_s, jnp.int32), jnp.uint32)

            @pl.when(wid == 0)
            def _():
                selv[...] = sel
                rv[...] = jnp.full((L,), rank_s, jnp.int32)
                pltpu.sync_copy(selv, selo_hbm)
                pltpu.sync_copy(rv, ranko_hbm)

        zero16 = jnp.zeros((L,), jnp.int32)

        @pl.loop(0, stride, unroll=8)
        def _(i):
            hist[pl.ds(i * L, L)] = zero16

        lane_base = lax.iota(jnp.int32, L) * stride
        ones = jnp.ones((L,), jnp.int32)

        def process(buf):
            @pl.loop(0, VECS // K)
            def _(g):
                b0 = g * (K * L)
                keys = [_key_u32(buf[pl.ds(b0 + k * L, L)]) for k in range(K)]
                if shift:
                    bs = [(key >> shift) & jnp.uint32(nb - 1) for key in keys]
                else:
                    bs = [key & jnp.uint32(nb - 1) for key in keys]
                addrs = [lane_base + plsc.bitcast(b, jnp.int32) for b in bs]
                if merge is None:
                    for addr in addrs:
                        plsc.addupdate_scatter(hist, [addr], ones)
                else:
                    ms = [(key >> match_shift) == sel for key in keys]
                    for addr, m in zip(addrs, ms):
                        plsc.addupdate_scatter(hist, [addr], ones, mask=m)

        @pl.loop(0, PAIRS)
        def _(j):
            c0 = j * 2
            wait(buf0, sem0)
            process(buf0)

            @pl.when(j < PAIRS - 1)
            def _():
                start(c0 + 2, buf0, sem0)

            wait(buf1, sem1)
            process(buf1)

            @pl.when(j < PAIRS - 1)
            def _():
                start(c0 + 3, buf1, sem1)

        @pl.loop(0, nb // L)
        def _(ci):
            a = hist[pl.ds(ci * L, L)]
            for lane in range(1, L):
                a = a + hist[pl.ds(lane * stride + ci * L, L)]
            orow[pl.ds(ci * L, L)] = a

        pltpu.sync_copy(orow, out_hbm.at[pl.ds(wid * nb, nb)])

    return pl.kernel(body, out_type=out_type, mesh=_mesh(),
                     scratch_types=scratch, compiler_params=_CP)


def _make_final(nb, out_shift):
    scratch = [
        pltpu.VMEM((ROWS_Q * nb,), jnp.int32),
        pltpu.VMEM((nb,), jnp.int32),
        pltpu.VMEM((L,), jnp.uint32),
        pltpu.VMEM((L,), jnp.int32),
        pltpu.VMEM((L,), jnp.float32),
    ]
    out_type = jax.ShapeDtypeStruct((L,), jnp.float32)

    def body(h_hbm, selp_hbm, rankp_hbm, out_hbm, hbuf, acc, selv, rv, ov):
        wid = lax.axis_index("s") * NC + lax.axis_index("c")

        @pl.when(wid == 0)
        def _():
            _reduce_rows(h_hbm, nb, hbuf, acc)
            pltpu.sync_copy(selp_hbm, selv)
            pltpu.sync_copy(rankp_hbm, rv)
            selp = jnp.max(plsc.bitcast(selv[...], jnp.int32))
            rankp = jnp.max(rv[...])
            bcnt, _ = _scan_select(acc, nb, rankp)
            keyv = ((jnp.full((L,), selp, jnp.int32) << out_shift)
                    | jnp.full((L,), bcnt, jnp.int32))
            uv = jnp.where(keyv < 0, keyv ^ MIN32, ~keyv)
            ov[...] = plsc.bitcast(uv, jnp.float32)
            pltpu.sync_copy(ov, out_hbm)

    return pl.kernel(body, out_type=out_type, mesh=_mesh(),
                     scratch_types=scratch, compiler_params=_CP)


_hist1 = _make_hist(shift=20, nb=NB12)
_hist2 = _make_hist(shift=8, nb=NB12, merge=(NB12, 20, 12))
_hist3 = _make_hist(shift=0, nb=NB3, merge=(NB12, 8, 12))
_final = _make_final(nb=NB3, out_shift=8)


def kernel(x):
    xf = jnp.transpose(x, (0, 2, 3, 1)).reshape(-1)
    sel0 = jnp.zeros((L,), jnp.uint32)
    r0 = jnp.full((L,), RANK, jnp.int32)
    h1 = _hist1(xf)
    h2, sel1, r1 = _hist2(xf, h1, sel0, r0)
    h3, sel2, r2 = _hist3(xf, h2, sel1, r1)
    v = _final(h3, sel2, r2)
    return x, v[0]


# speedup vs baseline: 4.8576x; 1.0366x over previous
import jax
import jax.numpy as jnp
import numpy as np
from jax import lax
from jax.experimental import pallas as pl
from jax.experimental.pallas import tpu as pltpu
from jax.experimental.pallas import tpu_sc as plsc

N = 32 * 192 * 56 * 56
RANK = int(N * 0.9995)
NC, NS, L = 2, 16, 16
NW = NC * NS
PER_W = N // NW
CHUNK = 2048
PAIRS = PER_W // (2 * CHUNK)
VECS = CHUNK // L
K = 16
NB12 = 4096
NB3 = 256
ROWS_Q = 8
MIN32 = np.int32(-(2 ** 31))

_CP = pltpu.CompilerParams(needs_layout_passes=False)


def _mesh():
    return plsc.VectorSubcoreMesh(core_axis_name="c", subcore_axis_name="s")


def _key_u32(xv):
    u = plsc.bitcast(xv, jnp.int32)
    sgn = u >> 31
    return plsc.bitcast(u ^ (sgn | MIN32), jnp.uint32)


def _reduce_rows(h_hbm, prev_nb, hbuf, acc):
    for q in range(NW // ROWS_Q):
        pltpu.sync_copy(h_hbm.at[pl.ds(q * ROWS_Q * prev_nb, ROWS_Q * prev_nb)],
                        hbuf)

        @pl.loop(0, prev_nb // (2 * L))
        def _(cj):
            avals = []
            for g in range(2):
                ci = cj * 2 + g
                a = hbuf[pl.ds(ci * L, L)]
                for r in range(1, ROWS_Q):
                    a = a + hbuf[pl.ds(r * prev_nb + ci * L, L)]
                if q > 0:
                    a = a + acc[pl.ds(ci * L, L)]
                avals.append(a)
            for g in range(2):
                acc[pl.ds((cj * 2 + g) * L, L)] = avals[g]


def _scan_select(acc, nbins, rank):
    def step(i, carry):
        bcnt, cbelow, cum = carry
        v = acc[pl.ds(i * L, L)]
        cs = plsc.cumsum(v) + cum
        m = cs < rank
        bcnt = bcnt + jnp.sum(m.astype(jnp.int32))
        cbelow = cbelow + jnp.sum(jnp.where(m, v, 0))
        cum = cum + jnp.sum(v)
        return bcnt, cbelow, cum

    z = jnp.int32(0)
    bcnt, cbelow, _ = lax.fori_loop(0, nbins // L, step, (z, z, z))
    return bcnt, cbelow


def _make_hist(shift, nb, merge=None):
    stride = nb + 1
    scratch = [
        pltpu.VMEM((CHUNK,), jnp.float32),
        pltpu.VMEM((CHUNK,), jnp.float32),
        pltpu.VMEM((L * stride,), jnp.int32),
        pltpu.VMEM((nb,), jnp.int32),
        pltpu.SemaphoreType.DMA,
        pltpu.SemaphoreType.DMA,
    ]
    if merge is not None:
        prev_nb, match_shift, out_shift = merge
        scratch += [
            pltpu.VMEM((ROWS_Q * prev_nb,), jnp.int32),
            pltpu.VMEM((prev_nb,), jnp.int32),
            pltpu.VMEM((L,), jnp.uint32),
            pltpu.VMEM((L,), jnp.int32),
        ]
        out_type = (jax.ShapeDtypeStruct((NW * nb,), jnp.int32),
                    jax.ShapeDtypeStruct((L,), jnp.uint32),
                    jax.ShapeDtypeStruct((L,), jnp.int32))
    else:
        out_type = jax.ShapeDtypeStruct((NW * nb,), jnp.int32)

    def body(x_hbm, *rest):
        if merge is not None:
            h_hbm, selp_hbm, rankp_hbm, out_hbm, selo_hbm, ranko_hbm = rest[:6]
            buf0, buf1, hist, orow, sem0, sem1, hbuf, acc, selv, rv = rest[6:]
        else:
            out_hbm = rest[0]
            buf0, buf1, hist, orow, sem0, sem1 = rest[1:]
        wid = lax.axis_index("s") * NC + lax.axis_index("c")
        base = wid * PER_W

        def start(c, buf, sem):
            pltpu.async_copy(x_hbm.at[pl.ds(base + c * CHUNK, CHUNK)], buf, sem)

        def wait(buf, sem):
            pltpu.make_async_copy(x_hbm.at[pl.ds(0, CHUNK)], buf, sem).wait()

        start(0, buf0, sem0)
        start(1, buf1, sem1)

        if merge is not None:
            _reduce_rows(h_hbm, prev_nb, hbuf, acc)
            pltpu.sync_copy(selp_hbm, selv)
            pltpu.sync_copy(rankp_hbm, rv)
            selp = jnp.max(plsc.bitcast(selv[...], jnp.int32))
            rankp = jnp.max(rv[...])
            bcnt, cbelow = _scan_select(acc, prev_nb, rankp)
            sel_s = (selp << out_shift) | bcnt
            rank_s = rankp - cbelow
            sel = plsc.bitcast(jnp.full((L,), sel_s, jnp.int32), jnp.uint32)

            @pl.when(wid == 0)
            def _():
                selv[...] = sel
                rv[...] = jnp.full((L,), rank_s, jnp.int32)
                pltpu.sync_copy(selv, selo_hbm)
                pltpu.sync_copy(rv, ranko_hbm)

        zero16 = jnp.zeros((L,), jnp.int32)

        @pl.loop(0, stride, unroll=8)
        def _(i):
            hist[pl.ds(i * L, L)] = zero16

        lane_base = lax.iota(jnp.int32, L) * stride
        ones = jnp.ones((L,), jnp.int32)

        def process(buf):
            @pl.loop(0, VECS // K)
            def _(g):
                b0 = g * (K * L)
                keys = [_key_u32(buf[pl.ds(b0 + k * L, L)]) for k in range(K)]
                if shift:
                    bs = [(key >> shift) & jnp.uint32(nb - 1) for key in keys]
                else:
                    bs = [key & jnp.uint32(nb - 1) for key in keys]
                addrs = [lane_base + plsc.bitcast(b, jnp.int32) for b in bs]
                if merge is None:
                    for addr in addrs:
                        plsc.addupdate_scatter(hist, [addr], ones)
                else:
                    ms = [(key >> match_shift) == sel for key in keys]
                    for addr, m in zip(addrs, ms):
                        plsc.addupdate_scatter(hist, [addr], ones, mask=m)

        @pl.loop(0, PAIRS)
        def _(j):
            c0 = j * 2
            wait(buf0, sem0)
            process(buf0)

            @pl.when(j < PAIRS - 1)
            def _():
                start(c0 + 2, buf0, sem0)

            wait(buf1, sem1)
            process(buf1)

            @pl.when(j < PAIRS - 1)
            def _():
                start(c0 + 3, buf1, sem1)

        @pl.loop(0, nb // L)
        def _(ci):
            a = hist[pl.ds(ci * L, L)]
            for lane in range(1, L):
                a = a + hist[pl.ds(lane * stride + ci * L, L)]
            orow[pl.ds(ci * L, L)] = a

        pltpu.sync_copy(orow, out_hbm.at[pl.ds(wid * nb, nb)])

    return pl.kernel(body, out_type=out_type, mesh=_mesh(),
                     scratch_types=scratch, compiler_params=_CP)


def _make_final(nb, out_shift):
    scratch = [
        pltpu.VMEM((ROWS_Q * nb,), jnp.int32),
        pltpu.VMEM((nb,), jnp.int32),
        pltpu.VMEM((L,), jnp.uint32),
        pltpu.VMEM((L,), jnp.int32),
        pltpu.VMEM((L,), jnp.float32),
    ]
    out_type = jax.ShapeDtypeStruct((L,), jnp.float32)

    def body(h_hbm, selp_hbm, rankp_hbm, out_hbm, hbuf, acc, selv, rv, ov):
        wid = lax.axis_index("s") * NC + lax.axis_index("c")

        @pl.when(wid == 0)
        def _():
            _reduce_rows(h_hbm, nb, hbuf, acc)
            pltpu.sync_copy(selp_hbm, selv)
            pltpu.sync_copy(rankp_hbm, rv)
            selp = jnp.max(plsc.bitcast(selv[...], jnp.int32))
            rankp = jnp.max(rv[...])
            bcnt, _ = _scan_select(acc, nb, rankp)
            keyv = ((jnp.full((L,), selp, jnp.int32) << out_shift)
                    | jnp.full((L,), bcnt, jnp.int32))
            uv = jnp.where(keyv < 0, keyv ^ MIN32, ~keyv)
            ov[...] = plsc.bitcast(uv, jnp.float32)
            pltpu.sync_copy(ov, out_hbm)

    return pl.kernel(body, out_type=out_type, mesh=_mesh(),
                     scratch_types=scratch, compiler_params=_CP)


_hist1 = _make_hist(shift=20, nb=NB12)
_hist2 = _make_hist(shift=8, nb=NB12, merge=(NB12, 20, 12))
_hist3 = _make_hist(shift=0, nb=NB3, merge=(NB12, 8, 12))
_final = _make_final(nb=NB3, out_shift=8)


def kernel(x):
    xf = jnp.transpose(x, (0, 2, 3, 1)).reshape(-1)
    sel0 = jnp.zeros((L,), jnp.uint32)
    r0 = jnp.full((L,), RANK, jnp.int32)
    h1 = _hist1(xf)
    h2, sel1, r1 = _hist2(xf, h1, sel0, r0)
    h3, sel2, r2 = _hist3(xf, h2, sel1, r1)
    v = _final(h3, sel2, r2)
    return x, v[0]


# speedup vs baseline: 5.5312x; 1.1387x over previous
import jax
import jax.numpy as jnp
import numpy as np
from jax import lax
from jax.experimental import pallas as pl
from jax.experimental.pallas import tpu as pltpu
from jax.experimental.pallas import tpu_sc as plsc

N = 32 * 192 * 56 * 56
RANK = int(N * 0.9995)
NC, NS, L = 2, 16, 16
NW = NC * NS
PER_W = N // NW
CHUNK = 3072
PAIRS = PER_W // (2 * CHUNK)
VECS = CHUNK // L
K = 16
NB12 = 4096
NB3 = 256
ROWS_Q = 8
MIN32 = np.int32(-(2 ** 31))

_CP = pltpu.CompilerParams(needs_layout_passes=False)


def _mesh():
    return plsc.VectorSubcoreMesh(core_axis_name="c", subcore_axis_name="s")


def _key_u32(xv):
    u = plsc.bitcast(xv, jnp.int32)
    sgn = u >> 31
    return plsc.bitcast(u ^ (sgn | MIN32), jnp.uint32)


def _reduce_rows(h_hbm, prev_nb, hbuf, acc):
    for q in range(NW // ROWS_Q):
        pltpu.sync_copy(h_hbm.at[pl.ds(q * ROWS_Q * prev_nb, ROWS_Q * prev_nb)],
                        hbuf)

        @pl.loop(0, prev_nb // (2 * L))
        def _(cj):
            avals = []
            for g in range(2):
                ci = cj * 2 + g
                a = hbuf[pl.ds(ci * L, L)]
                for r in range(1, ROWS_Q):
                    a = a + hbuf[pl.ds(r * prev_nb + ci * L, L)]
                if q > 0:
                    a = a + acc[pl.ds(ci * L, L)]
                avals.append(a)
            for g in range(2):
                acc[pl.ds((cj * 2 + g) * L, L)] = avals[g]


def _scan_select(acc, nbins, rank):
    def step(i, carry):
        bcnt, cbelow, cum = carry
        v = acc[pl.ds(i * L, L)]
        cs = plsc.cumsum(v) + cum
        m = cs < rank
        bcnt = bcnt + jnp.sum(m.astype(jnp.int32))
        cbelow = cbelow + jnp.sum(jnp.where(m, v, 0))
        cum = cum + jnp.sum(v)
        return bcnt, cbelow, cum

    z = jnp.int32(0)
    bcnt, cbelow, _ = lax.fori_loop(0, nbins // L, step, (z, z, z))
    return bcnt, cbelow


def _make_hist(shift, nb, merge=None):
    stride = nb + 1
    scratch = [
        pltpu.VMEM((CHUNK,), jnp.float32),
        pltpu.VMEM((CHUNK,), jnp.float32),
        pltpu.VMEM((L * stride,), jnp.int32),
        pltpu.VMEM((nb,), jnp.int32),
        pltpu.SemaphoreType.DMA,
        pltpu.SemaphoreType.DMA,
    ]
    if merge is not None:
        prev_nb, match_shift, out_shift = merge
        scratch += [
            pltpu.VMEM((ROWS_Q * prev_nb,), jnp.int32),
            pltpu.VMEM((prev_nb,), jnp.int32),
            pltpu.VMEM((L,), jnp.uint32),
            pltpu.VMEM((L,), jnp.int32),
        ]
        out_type = (jax.ShapeDtypeStruct((NW * nb,), jnp.int32),
                    jax.ShapeDtypeStruct((L,), jnp.uint32),
                    jax.ShapeDtypeStruct((L,), jnp.int32))
    else:
        out_type = jax.ShapeDtypeStruct((NW * nb,), jnp.int32)

    def body(x_hbm, *rest):
        if merge is not None:
            h_hbm, selp_hbm, rankp_hbm, out_hbm, selo_hbm, ranko_hbm = rest[:6]
            buf0, buf1, hist, orow, sem0, sem1, hbuf, acc, selv, rv = rest[6:]
        else:
            out_hbm = rest[0]
            buf0, buf1, hist, orow, sem0, sem1 = rest[1:]
        wid = lax.axis_index("s") * NC + lax.axis_index("c")
        base = wid * PER_W

        def start(c, buf, sem):
            pltpu.async_copy(x_hbm.at[pl.ds(base + c * CHUNK, CHUNK)], buf, sem)

        def wait(buf, sem):
            pltpu.make_async_copy(x_hbm.at[pl.ds(0, CHUNK)], buf, sem).wait()

        start(0, buf0, sem0)
        start(1, buf1, sem1)

        if merge is not None:
            _reduce_rows(h_hbm, prev_nb, hbuf, acc)
            pltpu.sync_copy(selp_hbm, selv)
            pltpu.sync_copy(rankp_hbm, rv)
            selp = jnp.max(plsc.bitcast(selv[...], jnp.int32))
            rankp = jnp.max(rv[...])
            bcnt, cbelow = _scan_select(acc, prev_nb, rankp)
            sel_s = (selp << out_shift) | bcnt
            rank_s = rankp - cbelow
            sel = plsc.bitcast(jnp.full((L,), sel_s, jnp.int32), jnp.uint32)

            @pl.when(wid == 0)
            def _():
                selv[...] = sel
                rv[...] = jnp.full((L,), rank_s, jnp.int32)
                pltpu.sync_copy(selv, selo_hbm)
                pltpu.sync_copy(rv, ranko_hbm)

        zero16 = jnp.zeros((L,), jnp.int32)

        @pl.loop(0, stride, unroll=8)
        def _(i):
            hist[pl.ds(i * L, L)] = zero16

        lane_base = lax.iota(jnp.int32, L) * stride
        ones = jnp.ones((L,), jnp.int32)

        def process(buf):
            @pl.loop(0, VECS // K)
            def _(g):
                b0 = g * (K * L)
                keys = [_key_u32(buf[pl.ds(b0 + k * L, L)]) for k in range(K)]
                if shift:
                    bs = [(key >> shift) & jnp.uint32(nb - 1) for key in keys]
                else:
                    bs = [key & jnp.uint32(nb - 1) for key in keys]
                addrs = [lane_base + plsc.bitcast(b, jnp.int32) for b in bs]
                if merge is None:
                    for addr in addrs:
                        plsc.addupdate_scatter(hist, [addr], ones)
                else:
                    ms = [(key >> match_shift) == sel for key in keys]
                    for addr, m in zip(addrs, ms):
                        plsc.addupdate_scatter(hist, [addr], ones, mask=m)

        @pl.loop(0, PAIRS)
        def _(j):
            c0 = j * 2
            wait(buf0, sem0)
            process(buf0)

            @pl.when(j < PAIRS - 1)
            def _():
                start(c0 + 2, buf0, sem0)

            wait(buf1, sem1)
            process(buf1)

            @pl.when(j < PAIRS - 1)
            def _():
                start(c0 + 3, buf1, sem1)

        @pl.loop(0, nb // L)
        def _(ci):
            a = hist[pl.ds(ci * L, L)]
            for lane in range(1, L):
                a = a + hist[pl.ds(lane * stride + ci * L, L)]
            orow[pl.ds(ci * L, L)] = a

        pltpu.sync_copy(orow, out_hbm.at[pl.ds(wid * nb, nb)])

    return pl.kernel(body, out_type=out_type, mesh=_mesh(),
                     scratch_types=scratch, compiler_params=_CP)


def _make_final(nb, out_shift):
    scratch = [
        pltpu.VMEM((ROWS_Q * nb,), jnp.int32),
        pltpu.VMEM((nb,), jnp.int32),
        pltpu.VMEM((L,), jnp.uint32),
        pltpu.VMEM((L,), jnp.int32),
        pltpu.VMEM((L,), jnp.float32),
    ]
    out_type = jax.ShapeDtypeStruct((L,), jnp.float32)

    def body(h_hbm, selp_hbm, rankp_hbm, out_hbm, hbuf, acc, selv, rv, ov):
        wid = lax.axis_index("s") * NC + lax.axis_index("c")

        @pl.when(wid == 0)
        def _():
            _reduce_rows(h_hbm, nb, hbuf, acc)
            pltpu.sync_copy(selp_hbm, selv)
            pltpu.sync_copy(rankp_hbm, rv)
            selp = jnp.max(plsc.bitcast(selv[...], jnp.int32))
            rankp = jnp.max(rv[...])
            bcnt, _ = _scan_select(acc, nb, rankp)
            keyv = ((jnp.full((L,), selp, jnp.int32) << out_shift)
                    | jnp.full((L,), bcnt, jnp.int32))
            uv = jnp.where(keyv < 0, keyv ^ MIN32, ~keyv)
            ov[...] = plsc.bitcast(uv, jnp.float32)
            pltpu.sync_copy(ov, out_hbm)

    return pl.kernel(body, out_type=out_type, mesh=_mesh(),
                     scratch_types=scratch, compiler_params=_CP)


_hist1 = _make_hist(shift=20, nb=NB12)
_hist2 = _make_hist(shift=8, nb=NB12, merge=(NB12, 20, 12))
_hist3 = _make_hist(shift=0, nb=NB3, merge=(NB12, 8, 12))
_final = _make_final(nb=NB3, out_shift=8)


def kernel(x):
    xf = jnp.transpose(x, (0, 2, 3, 1)).reshape(-1)
    sel0 = jnp.zeros((L,), jnp.uint32)
    r0 = jnp.full((L,), RANK, jnp.int32)
    h1 = _hist1(xf)
    h2, sel1, r1 = _hist2(xf, h1, sel0, r0)
    h3, sel2, r2 = _hist3(xf, h2, sel1, r1)
    v = _final(h3, sel2, r2)
    return x, v[0]


# speedup vs baseline: 6.4720x; 1.1701x over previous
import jax
import jax.numpy as jnp
import numpy as np
from jax import lax
from jax.experimental import pallas as pl
from jax.experimental.pallas import tpu as pltpu
from jax.experimental.pallas import tpu_sc as plsc

N = 32 * 192 * 56 * 56
RANK = int(N * 0.9995)
NC, NS, L = 2, 16, 16
NW = NC * NS
PER_W = N // NW
CHUNK = 6144
PAIRS = PER_W // (2 * CHUNK)
VECS = CHUNK // L
K = 16
NB12 = 4096
NB3 = 256
ROWS_Q = 8
MIN32 = np.int32(-(2 ** 31))

_CP = pltpu.CompilerParams(needs_layout_passes=False)


def _mesh():
    return plsc.VectorSubcoreMesh(core_axis_name="c", subcore_axis_name="s")


def _key_u32(xv):
    u = plsc.bitcast(xv, jnp.int32)
    sgn = u >> 31
    return plsc.bitcast(u ^ (sgn | MIN32), jnp.uint32)


def _reduce_rows(h_hbm, prev_nb, hbuf, acc):
    for q in range(NW // ROWS_Q):
        pltpu.sync_copy(h_hbm.at[pl.ds(q * ROWS_Q * prev_nb, ROWS_Q * prev_nb)],
                        hbuf)

        @pl.loop(0, prev_nb // (2 * L))
        def _(cj):
            avals = []
            for g in range(2):
                ci = cj * 2 + g
                a = hbuf[pl.ds(ci * L, L)]
                for r in range(1, ROWS_Q):
                    a = a + hbuf[pl.ds(r * prev_nb + ci * L, L)]
                if q > 0:
                    a = a + acc[pl.ds(ci * L, L)]
                avals.append(a)
            for g in range(2):
                acc[pl.ds((cj * 2 + g) * L, L)] = avals[g]


def _scan_select(acc, nbins, rank):
    def step(i, carry):
        bcnt, cbelow, cum = carry
        v = acc[pl.ds(i * L, L)]
        cs = plsc.cumsum(v) + cum
        m = cs < rank
        bcnt = bcnt + jnp.sum(m.astype(jnp.int32))
        cbelow = cbelow + jnp.sum(jnp.where(m, v, 0))
        cum = cum + jnp.sum(v)
        return bcnt, cbelow, cum

    z = jnp.int32(0)
    bcnt, cbelow, _ = lax.fori_loop(0, nbins // L, step, (z, z, z))
    return bcnt, cbelow


def _make_hist(shift, nb, merge=None):
    stride = nb + 1
    scratch = [
        pltpu.VMEM((CHUNK,), jnp.float32),
        pltpu.VMEM((CHUNK,), jnp.float32),
        pltpu.VMEM((L * stride,), jnp.int32),
        pltpu.VMEM((nb,), jnp.int32),
        pltpu.SemaphoreType.DMA,
        pltpu.SemaphoreType.DMA,
    ]
    if merge is not None:
        prev_nb, match_shift, out_shift = merge
        scratch += [
            pltpu.VMEM((ROWS_Q * prev_nb,), jnp.int32),
            pltpu.VMEM((prev_nb,), jnp.int32),
            pltpu.VMEM((L,), jnp.uint32),
            pltpu.VMEM((L,), jnp.int32),
        ]
        out_type = (jax.ShapeDtypeStruct((NW * nb,), jnp.int32),
                    jax.ShapeDtypeStruct((L,), jnp.uint32),
                    jax.ShapeDtypeStruct((L,), jnp.int32))
    else:
        out_type = jax.ShapeDtypeStruct((NW * nb,), jnp.int32)

    def body(x_hbm, *rest):
        if merge is not None:
            h_hbm, selp_hbm, rankp_hbm, out_hbm, selo_hbm, ranko_hbm = rest[:6]
            buf0, buf1, hist, orow, sem0, sem1, hbuf, acc, selv, rv = rest[6:]
        else:
            out_hbm = rest[0]
            buf0, buf1, hist, orow, sem0, sem1 = rest[1:]
        wid = lax.axis_index("s") * NC + lax.axis_index("c")
        base = wid * PER_W

        def start(c, buf, sem):
            pltpu.async_copy(x_hbm.at[pl.ds(base + c * CHUNK, CHUNK)], buf, sem)

        def wait(buf, sem):
            pltpu.make_async_copy(x_hbm.at[pl.ds(0, CHUNK)], buf, sem).wait()

        start(0, buf0, sem0)
        start(1, buf1, sem1)

        if merge is not None:
            _reduce_rows(h_hbm, prev_nb, hbuf, acc)
            pltpu.sync_copy(selp_hbm, selv)
            pltpu.sync_copy(rankp_hbm, rv)
            selp = jnp.max(plsc.bitcast(selv[...], jnp.int32))
            rankp = jnp.max(rv[...])
            bcnt, cbelow = _scan_select(acc, prev_nb, rankp)
            sel_s = (selp << out_shift) | bcnt
            rank_s = rankp - cbelow
            sel = plsc.bitcast(jnp.full((L,), sel_s, jnp.int32), jnp.uint32)

            @pl.when(wid == 0)
            def _():
                selv[...] = sel
                rv[...] = jnp.full((L,), rank_s, jnp.int32)
                pltpu.sync_copy(selv, selo_hbm)
                pltpu.sync_copy(rv, ranko_hbm)

        zero16 = jnp.zeros((L,), jnp.int32)

        @pl.loop(0, stride, unroll=8)
        def _(i):
            hist[pl.ds(i * L, L)] = zero16

        lane_base = lax.iota(jnp.int32, L) * stride
        ones = jnp.ones((L,), jnp.int32)

        def process(buf):
            @pl.loop(0, VECS // K)
            def _(g):
                b0 = g * (K * L)
                keys = [_key_u32(buf[pl.ds(b0 + k * L, L)]) for k in range(K)]
                if shift:
                    bs = [(key >> shift) & jnp.uint32(nb - 1) for key in keys]
                else:
                    bs = [key & jnp.uint32(nb - 1) for key in keys]
                addrs = [lane_base + plsc.bitcast(b, jnp.int32) for b in bs]
                if merge is None:
                    for addr in addrs:
                        plsc.addupdate_scatter(hist, [addr], ones)
                else:
                    ms = [(key >> match_shift) == sel for key in keys]
                    for addr, m in zip(addrs, ms):
                        plsc.addupdate_scatter(hist, [addr], ones, mask=m)

        @pl.loop(0, PAIRS)
        def _(j):
            c0 = j * 2
            wait(buf0, sem0)
            process(buf0)

            @pl.when(j < PAIRS - 1)
            def _():
                start(c0 + 2, buf0, sem0)

            wait(buf1, sem1)
            process(buf1)

            @pl.when(j < PAIRS - 1)
            def _():
                start(c0 + 3, buf1, sem1)

        @pl.loop(0, nb // L)
        def _(ci):
            a = hist[pl.ds(ci * L, L)]
            for lane in range(1, L):
                a = a + hist[pl.ds(lane * stride + ci * L, L)]
            orow[pl.ds(ci * L, L)] = a

        pltpu.sync_copy(orow, out_hbm.at[pl.ds(wid * nb, nb)])

    return pl.kernel(body, out_type=out_type, mesh=_mesh(),
                     scratch_types=scratch, compiler_params=_CP)


def _make_final(nb, out_shift):
    scratch = [
        pltpu.VMEM((ROWS_Q * nb,), jnp.int32),
        pltpu.VMEM((nb,), jnp.int32),
        pltpu.VMEM((L,), jnp.uint32),
        pltpu.VMEM((L,), jnp.int32),
        pltpu.VMEM((L,), jnp.float32),
    ]
    out_type = jax.ShapeDtypeStruct((L,), jnp.float32)

    def body(h_hbm, selp_hbm, rankp_hbm, out_hbm, hbuf, acc, selv, rv, ov):
        wid = lax.axis_index("s") * NC + lax.axis_index("c")

        @pl.when(wid == 0)
        def _():
            _reduce_rows(h_hbm, nb, hbuf, acc)
            pltpu.sync_copy(selp_hbm, selv)
            pltpu.sync_copy(rankp_hbm, rv)
            selp = jnp.max(plsc.bitcast(selv[...], jnp.int32))
            rankp = jnp.max(rv[...])
            bcnt, _ = _scan_select(acc, nb, rankp)
            keyv = ((jnp.full((L,), selp, jnp.int32) << out_shift)
                    | jnp.full((L,), bcnt, jnp.int32))
            uv = jnp.where(keyv < 0, keyv ^ MIN32, ~keyv)
            ov[...] = plsc.bitcast(uv, jnp.float32)
            pltpu.sync_copy(ov, out_hbm)

    return pl.kernel(body, out_type=out_type, mesh=_mesh(),
                     scratch_types=scratch, compiler_params=_CP)


_hist1 = _make_hist(shift=20, nb=NB12)
_hist2 = _make_hist(shift=8, nb=NB12, merge=(NB12, 20, 12))
_hist3 = _make_hist(shift=0, nb=NB3, merge=(NB12, 8, 12))
_final = _make_final(nb=NB3, out_shift=8)


def kernel(x):
    xf = jnp.transpose(x, (0, 2, 3, 1)).reshape(-1)
    sel0 = jnp.zeros((L,), jnp.uint32)
    r0 = jnp.full((L,), RANK, jnp.int32)
    h1 = _hist1(xf)
    h2, sel1, r1 = _hist2(xf, h1, sel0, r0)
    h3, sel2, r2 = _hist3(xf, h2, sel1, r1)
    v = _final(h3, sel2, r2)
    return x, v[0]


# speedup vs baseline: 6.6304x; 1.0245x over previous
import jax
import jax.numpy as jnp
import numpy as np
from jax import lax
from jax.experimental import pallas as pl
from jax.experimental.pallas import tpu as pltpu
from jax.experimental.pallas import tpu_sc as plsc

N = 32 * 192 * 56 * 56
RANK = int(N * 0.9995)
NC, NS, L = 2, 16, 16
NW = NC * NS
PER_W = N // NW
CHUNK = 7168
PAIRS = PER_W // (2 * CHUNK)
VECS = CHUNK // L
K = 16
NB12 = 4096
NB3 = 256
ROWS_Q = 8
MIN32 = np.int32(-(2 ** 31))

_CP = pltpu.CompilerParams(needs_layout_passes=False)


def _mesh():
    return plsc.VectorSubcoreMesh(core_axis_name="c", subcore_axis_name="s")


def _key_u32(xv):
    u = plsc.bitcast(xv, jnp.int32)
    sgn = u >> 31
    return plsc.bitcast(u ^ (sgn | MIN32), jnp.uint32)


def _reduce_rows(h_hbm, prev_nb, hbuf, acc):
    for q in range(NW // ROWS_Q):
        pltpu.sync_copy(h_hbm.at[pl.ds(q * ROWS_Q * prev_nb, ROWS_Q * prev_nb)],
                        hbuf)

        @pl.loop(0, prev_nb // (2 * L))
        def _(cj):
            avals = []
            for g in range(2):
                ci = cj * 2 + g
                a = hbuf[pl.ds(ci * L, L)]
                for r in range(1, ROWS_Q):
                    a = a + hbuf[pl.ds(r * prev_nb + ci * L, L)]
                if q > 0:
                    a = a + acc[pl.ds(ci * L, L)]
                avals.append(a)
            for g in range(2):
                acc[pl.ds((cj * 2 + g) * L, L)] = avals[g]


def _scan_select(acc, nbins, rank):
    def step(i, carry):
        bcnt, cbelow, cum = carry
        v = acc[pl.ds(i * L, L)]
        cs = plsc.cumsum(v) + cum
        m = cs < rank
        bcnt = bcnt + jnp.sum(m.astype(jnp.int32))
        cbelow = cbelow + jnp.sum(jnp.where(m, v, 0))
        cum = cum + jnp.sum(v)
        return bcnt, cbelow, cum

    z = jnp.int32(0)
    bcnt, cbelow, _ = lax.fori_loop(0, nbins // L, step, (z, z, z))
    return bcnt, cbelow


def _make_hist(shift, nb, merge=None):
    stride = nb + 1
    scratch = [
        pltpu.VMEM((CHUNK,), jnp.float32),
        pltpu.VMEM((CHUNK,), jnp.float32),
        pltpu.VMEM((L * stride,), jnp.int32),
        pltpu.VMEM((nb,), jnp.int32),
        pltpu.SemaphoreType.DMA,
        pltpu.SemaphoreType.DMA,
    ]
    if merge is not None:
        prev_nb, match_shift, out_shift = merge
        scratch += [
            pltpu.VMEM((ROWS_Q * prev_nb,), jnp.int32),
            pltpu.VMEM((prev_nb,), jnp.int32),
            pltpu.VMEM((L,), jnp.uint32),
            pltpu.VMEM((L,), jnp.int32),
        ]
        out_type = (jax.ShapeDtypeStruct((NW * nb,), jnp.int32),
                    jax.ShapeDtypeStruct((L,), jnp.uint32),
                    jax.ShapeDtypeStruct((L,), jnp.int32))
    else:
        out_type = jax.ShapeDtypeStruct((NW * nb,), jnp.int32)

    def body(x_hbm, *rest):
        if merge is not None:
            h_hbm, selp_hbm, rankp_hbm, out_hbm, selo_hbm, ranko_hbm = rest[:6]
            buf0, buf1, hist, orow, sem0, sem1, hbuf, acc, selv, rv = rest[6:]
        else:
            out_hbm = rest[0]
            buf0, buf1, hist, orow, sem0, sem1 = rest[1:]
        wid = lax.axis_index("s") * NC + lax.axis_index("c")
        base = wid * PER_W

        def start(c, buf, sem):
            pltpu.async_copy(x_hbm.at[pl.ds(base + c * CHUNK, CHUNK)], buf, sem)

        def wait(buf, sem):
            pltpu.make_async_copy(x_hbm.at[pl.ds(0, CHUNK)], buf, sem).wait()

        start(0, buf0, sem0)
        start(1, buf1, sem1)

        if merge is not None:
            _reduce_rows(h_hbm, prev_nb, hbuf, acc)
            pltpu.sync_copy(selp_hbm, selv)
            pltpu.sync_copy(rankp_hbm, rv)
            selp = jnp.max(plsc.bitcast(selv[...], jnp.int32))
            rankp = jnp.max(rv[...])
            bcnt, cbelow = _scan_select(acc, prev_nb, rankp)
            sel_s = (selp << out_shift) | bcnt
            rank_s = rankp - cbelow
            sel = plsc.bitcast(jnp.full((L,), sel_s, jnp.int32), jnp.uint32)

            @pl.when(wid == 0)
            def _():
                selv[...] = sel
                rv[...] = jnp.full((L,), rank_s, jnp.int32)
                pltpu.sync_copy(selv, selo_hbm)
                pltpu.sync_copy(rv, ranko_hbm)

        zero16 = jnp.zeros((L,), jnp.int32)

        @pl.loop(0, stride, unroll=8)
        def _(i):
            hist[pl.ds(i * L, L)] = zero16

        lane_base = lax.iota(jnp.int32, L) * stride
        ones = jnp.ones((L,), jnp.int32)

        def process(buf):
            @pl.loop(0, VECS // K)
            def _(g):
                b0 = g * (K * L)
                keys = [_key_u32(buf[pl.ds(b0 + k * L, L)]) for k in range(K)]
                if shift:
                    bs = [(key >> shift) & jnp.uint32(nb - 1) for key in keys]
                else:
                    bs = [key & jnp.uint32(nb - 1) for key in keys]
                addrs = [lane_base + plsc.bitcast(b, jnp.int32) for b in bs]
                if merge is None:
                    for addr in addrs:
                        plsc.addupdate_scatter(hist, [addr], ones)
                else:
                    ms = [(key >> match_shift) == sel for key in keys]
                    for addr, m in zip(addrs, ms):
                        plsc.addupdate_scatter(hist, [addr], ones, mask=m)

        @pl.loop(0, PAIRS)
        def _(j):
            c0 = j * 2
            wait(buf0, sem0)
            process(buf0)

            @pl.when(j < PAIRS - 1)
            def _():
                start(c0 + 2, buf0, sem0)

            wait(buf1, sem1)
            process(buf1)

            @pl.when(j < PAIRS - 1)
            def _():
                start(c0 + 3, buf1, sem1)

        @pl.loop(0, nb // L)
        def _(ci):
            a = hist[pl.ds(ci * L, L)]
            for lane in range(1, L):
                a = a + hist[pl.ds(lane * stride + ci * L, L)]
            orow[pl.ds(ci * L, L)] = a

        pltpu.sync_copy(orow, out_hbm.at[pl.ds(wid * nb, nb)])

    return pl.kernel(body, out_type=out_type, mesh=_mesh(),
                     scratch_types=scratch, compiler_params=_CP)


def _make_final(nb, out_shift):
    scratch = [
        pltpu.VMEM((ROWS_Q * nb,), jnp.int32),
        pltpu.VMEM((nb,), jnp.int32),
        pltpu.VMEM((L,), jnp.uint32),
        pltpu.VMEM((L,), jnp.int32),
        pltpu.VMEM((L,), jnp.float32),
    ]
    out_type = jax.ShapeDtypeStruct((L,), jnp.float32)

    def body(h_hbm, selp_hbm, rankp_hbm, out_hbm, hbuf, acc, selv, rv, ov):
        wid = lax.axis_index("s") * NC + lax.axis_index("c")

        @pl.when(wid == 0)
        def _():
            _reduce_rows(h_hbm, nb, hbuf, acc)
            pltpu.sync_copy(selp_hbm, selv)
            pltpu.sync_copy(rankp_hbm, rv)
            selp = jnp.max(plsc.bitcast(selv[...], jnp.int32))
            rankp = jnp.max(rv[...])
            bcnt, _ = _scan_select(acc, nb, rankp)
            keyv = ((jnp.full((L,), selp, jnp.int32) << out_shift)
                    | jnp.full((L,), bcnt, jnp.int32))
            uv = jnp.where(keyv < 0, keyv ^ MIN32, ~keyv)
            ov[...] = plsc.bitcast(uv, jnp.float32)
            pltpu.sync_copy(ov, out_hbm)

    return pl.kernel(body, out_type=out_type, mesh=_mesh(),
                     scratch_types=scratch, compiler_params=_CP)


_hist1 = _make_hist(shift=20, nb=NB12)
_hist2 = _make_hist(shift=8, nb=NB12, merge=(NB12, 20, 12))
_hist3 = _make_hist(shift=0, nb=NB3, merge=(NB12, 8, 12))
_final = _make_final(nb=NB3, out_shift=8)


def kernel(x):
    xf = jnp.transpose(x, (0, 2, 3, 1)).reshape(-1)
    sel0 = jnp.zeros((L,), jnp.uint32)
    r0 = jnp.full((L,), RANK, jnp.int32)
    h1 = _hist1(xf)
    h2, sel1, r1 = _hist2(xf, h1, sel0, r0)
    h3, sel2, r2 = _hist3(xf, h2, sel1, r1)
    v = _final(h3, sel2, r2)
    return x, v[0]
